# baseline (speedup 1.0000x reference)
import jax
import jax.numpy as jnp
from jax import lax
from jax.experimental import pallas as pl
from jax.experimental.pallas import tpu as pltpu
from jax.experimental.pallas import tpu_sc as plsc

B = 16384
L = 200
EMB = 64
HID = 32
HALF_L = L // 2
V = 1000000
PBLK = 8192
NPBLK = (V + PBLK - 1) // PBLK

_info = plsc.get_sparse_core_info()
NC, NS = _info.num_cores, _info.num_subcores
NW = NC * NS
ROWS_W = B // NW
CHUNK = 64
NCHUNK = ROWS_W // CHUNK


def _fold_body(a_ref, w_ref, o_ref):
    o_ref[...] = lax.dot_general(
        a_ref[...], w_ref[...], (((0,), (0,)), ((), ())),
        precision=lax.Precision.HIGHEST,
        preferred_element_type=jnp.float32)


def _fold_table(table, W1):
    tT = table.T
    return pl.pallas_call(
        _fold_body,
        grid=(NPBLK,),
        in_specs=[
            pl.BlockSpec((EMB, PBLK), lambda i: (0, i)),
            pl.BlockSpec((EMB, HID), lambda i: (0, 0)),
        ],
        out_specs=pl.BlockSpec((PBLK, HID), lambda i: (i, 0)),
        out_shape=jax.ShapeDtypeStruct((V, HID), jnp.float32),
    )(tT, W1)


def _sc_pool_body(xr_hbm, t1_hbm, psum_hbm, idx_v, rows0, rows1, out_v,
                  sem_a, sem_b, sem_i):
    cc = lax.axis_index("c")
    ss = lax.axis_index("s")
    wid = ss * NC + cc
    rbase = wid * ROWS_W

    def idx_copy(ch, ib):
        return pltpu.make_async_copy(
            xr_hbm.at[pl.ds((rbase + ch * CHUNK) * 2, CHUNK * 2)],
            idx_v.at[ib], sem_i)

    def row_copies(cb, r2, rowbuf, sem):
        c0 = pltpu.make_async_copy(
            t1_hbm.at[idx_v.at[cb, 2 * r2]],
            rowbuf.at[pl.ds(0, HALF_L)], sem)
        c1 = pltpu.make_async_copy(
            t1_hbm.at[idx_v.at[cb, 2 * r2 + 1]],
            rowbuf.at[pl.ds(HALF_L, HALF_L)], sem)
        return c0, c1

    def start_row(cb, r2, rowbuf, sem):
        c0, c1 = row_copies(cb, r2, rowbuf, sem)
        c0.start()
        c1.start()

    def wait_row(cb, r2, rowbuf, sem):
        c0, c1 = row_copies(cb, r2, rowbuf, sem)
        c0.wait()
        c1.wait()

    def reduce_row(rowbuf, r2):
        def red(i, accs):
            res = list(accs)
            for u in range(8):
                r = i * 8 + u
                for c in range(2):
                    res[c] = res[c] + rowbuf[r, pl.ds(c * 16, 16)]
            return tuple(res)

        accs = lax.fori_loop(
            0, L // 8, red,
            tuple(jnp.zeros((16,), jnp.float32) for _ in range(2)))
        for c in range(2):
            out_v[r2, pl.ds(c * 16, 16)] = accs[c]

    idx_copy(0, 0).start()
    idx_copy(0, 0).wait()
    idx_copy(1, 1).start()
    start_row(0, 0, rows0, sem_a)

    for ch in range(NCHUNK):
        cb = ch & 1
        cbase = rbase + ch * CHUNK

        def jbody(j, _):
            start_row(cb, 2 * j + 1, rows1, sem_b)
            wait_row(cb, 2 * j, rows0, sem_a)
            reduce_row(rows0, 2 * j)

            @pl.when(j < CHUNK // 2 - 1)
            def _():
                start_row(cb, 2 * j + 2, rows0, sem_a)

            wait_row(cb, 2 * j + 1, rows1, sem_b)
            reduce_row(rows1, 2 * j + 1)
            return 0

        lax.fori_loop(0, CHUNK // 2, jbody, 0)
        pltpu.sync_copy(out_v, psum_hbm.at[pl.ds(cbase, CHUNK)])
        if ch < NCHUNK - 1:
            idx_copy(ch + 1, 1 - cb).wait()
            if ch < NCHUNK - 2:
                idx_copy(ch + 2, cb).start()
            start_row(1 - cb, 0, rows0, sem_a)


def _sc_pool(xr, t1):
    kern = pl.kernel(
        _sc_pool_body,
        mesh=plsc.VectorSubcoreMesh(core_axis_name="c", subcore_axis_name="s"),
        out_type=jax.ShapeDtypeStruct((B, HID), jnp.float32),
        scratch_types=[
            pltpu.VMEM((2, 2 * CHUNK, HALF_L), jnp.int32),
            pltpu.VMEM((L, HID), jnp.float32),
            pltpu.VMEM((L, HID), jnp.float32),
            pltpu.VMEM((CHUNK, HID), jnp.float32),
            pltpu.SemaphoreType.DMA,
            pltpu.SemaphoreType.DMA,
            pltpu.SemaphoreType.DMA,
        ],
        compiler_params=pltpu.CompilerParams(use_tc_tiling_on_sc=False),
    )
    return kern(xr, t1)


def _tc_mlp_body(p_ref, b1_ref, w2_ref, b2_ref, o_ref):
    h = jnp.maximum(p_ref[...] * (1.0 / L) + b1_ref[...], 0.0)
    o_ref[...] = (
        jnp.dot(h, w2_ref[...], preferred_element_type=jnp.float32)
        + b2_ref[...])


def _tc_mlp(psum, b1, W2, b2):
    blk = 1024
    return pl.pallas_call(
        _tc_mlp_body,
        grid=(B // blk,),
        in_specs=[
            pl.BlockSpec((blk, HID), lambda i: (i, 0)),
            pl.BlockSpec((1, HID), lambda i: (0, 0)),
            pl.BlockSpec((HID, 1), lambda i: (0, 0)),
            pl.BlockSpec((1, 1), lambda i: (0, 0)),
        ],
        out_specs=pl.BlockSpec((blk, 1), lambda i: (i, 0)),
        out_shape=jax.ShapeDtypeStruct((B, 1), jnp.float32),
    )(psum, b1.reshape(1, HID), W2, b2.reshape(1, 1))


def kernel(x, table, W1, b1, W2, b2):
    xr = x.astype(jnp.int32).reshape(B * 2, HALF_L)
    t1 = _fold_table(table, W1)
    psum = _sc_pool(xr, t1)
    return _tc_mlp(psum, b1, W2, b2)


# speedup vs baseline: 1.1610x; 1.1610x over previous
import jax
import jax.numpy as jnp
from jax import lax
from jax.experimental import pallas as pl
from jax.experimental.pallas import tpu as pltpu
from jax.experimental.pallas import tpu_sc as plsc

B = 16384
L = 200
EMB = 64
HID = 32
HALF_L = L // 2
V = 1000000
PBLK = 8192
NPBLK = (V + PBLK - 1) // PBLK

_info = plsc.get_sparse_core_info()
NC, NS = _info.num_cores, _info.num_subcores
NW = NC * NS
ROWS_W = B // NW
CHUNK = 64
NCHUNK = ROWS_W // CHUNK


def _fold_body(a_ref, w_ref, o_ref):
    o_ref[...] = lax.dot_general(
        a_ref[...], w_ref[...], (((0,), (0,)), ((), ())),
        preferred_element_type=jnp.float32)


def _fold_table(table, W1):
    tT = table.T
    return pl.pallas_call(
        _fold_body,
        grid=(NPBLK,),
        in_specs=[
            pl.BlockSpec((EMB, PBLK), lambda i: (0, i)),
            pl.BlockSpec((EMB, HID), lambda i: (0, 0)),
        ],
        out_specs=pl.BlockSpec((PBLK, HID), lambda i: (i, 0)),
        out_shape=jax.ShapeDtypeStruct((V, HID), jnp.float32),
    )(tT, W1)


def _sc_pool_body(xr_hbm, t1_hbm, psum_hbm, idx_v, rows0, rows1, out_v,
                  sem_a, sem_b, sem_i):
    cc = lax.axis_index("c")
    ss = lax.axis_index("s")
    wid = ss * NC + cc
    rbase = wid * ROWS_W

    def idx_copy(ch, ib):
        return pltpu.make_async_copy(
            xr_hbm.at[pl.ds((rbase + ch * CHUNK) * 2, CHUNK * 2)],
            idx_v.at[ib], sem_i)

    def row_copies(cb, r2, rowbuf, sem):
        c0 = pltpu.make_async_copy(
            t1_hbm.at[idx_v.at[cb, 2 * r2]],
            rowbuf.at[pl.ds(0, HALF_L)], sem)
        c1 = pltpu.make_async_copy(
            t1_hbm.at[idx_v.at[cb, 2 * r2 + 1]],
            rowbuf.at[pl.ds(HALF_L, HALF_L)], sem)
        return c0, c1

    def start_row(cb, r2, rowbuf, sem):
        c0, c1 = row_copies(cb, r2, rowbuf, sem)
        c0.start()
        c1.start()

    def wait_row(cb, r2, rowbuf, sem):
        c0, c1 = row_copies(cb, r2, rowbuf, sem)
        c0.wait()
        c1.wait()

    def reduce_row(rowbuf, r2):
        def red(i, accs):
            res = list(accs)
            for u in range(8):
                r = i * 8 + u
                for c in range(2):
                    res[c] = res[c] + rowbuf[r, pl.ds(c * 16, 16)]
            return tuple(res)

        accs = lax.fori_loop(
            0, L // 8, red,
            tuple(jnp.zeros((16,), jnp.float32) for _ in range(2)))
        for c in range(2):
            out_v[r2, pl.ds(c * 16, 16)] = accs[c]

    idx_copy(0, 0).start()
    idx_copy(0, 0).wait()
    idx_copy(1, 1).start()
    start_row(0, 0, rows0, sem_a)

    for ch in range(NCHUNK):
        cb = ch & 1
        cbase = rbase + ch * CHUNK

        def jbody(j, _):
            start_row(cb, 2 * j + 1, rows1, sem_b)
            wait_row(cb, 2 * j, rows0, sem_a)
            reduce_row(rows0, 2 * j)

            @pl.when(j < CHUNK // 2 - 1)
            def _():
                start_row(cb, 2 * j + 2, rows0, sem_a)

            wait_row(cb, 2 * j + 1, rows1, sem_b)
            reduce_row(rows1, 2 * j + 1)
            return 0

        lax.fori_loop(0, CHUNK // 2, jbody, 0)
        pltpu.sync_copy(out_v, psum_hbm.at[pl.ds(cbase, CHUNK)])
        if ch < NCHUNK - 1:
            idx_copy(ch + 1, 1 - cb).wait()
            if ch < NCHUNK - 2:
                idx_copy(ch + 2, cb).start()
            start_row(1 - cb, 0, rows0, sem_a)


def _sc_pool(xr, t1):
    kern = pl.kernel(
        _sc_pool_body,
        mesh=plsc.VectorSubcoreMesh(core_axis_name="c", subcore_axis_name="s"),
        out_type=jax.ShapeDtypeStruct((B, HID), jnp.float32),
        scratch_types=[
            pltpu.VMEM((2, 2 * CHUNK, HALF_L), jnp.int32),
            pltpu.VMEM((L, HID), jnp.float32),
            pltpu.VMEM((L, HID), jnp.float32),
            pltpu.VMEM((CHUNK, HID), jnp.float32),
            pltpu.SemaphoreType.DMA,
            pltpu.SemaphoreType.DMA,
            pltpu.SemaphoreType.DMA,
        ],
        compiler_params=pltpu.CompilerParams(use_tc_tiling_on_sc=False),
    )
    return kern(xr, t1)


def _tc_mlp_body(p_ref, b1_ref, w2_ref, b2_ref, o_ref):
    h = jnp.maximum(p_ref[...] * (1.0 / L) + b1_ref[...], 0.0)
    o_ref[...] = (
        jnp.dot(h, w2_ref[...], preferred_element_type=jnp.float32)
        + b2_ref[...])


def _tc_mlp(psum, b1, W2, b2):
    blk = 1024
    return pl.pallas_call(
        _tc_mlp_body,
        grid=(B // blk,),
        in_specs=[
            pl.BlockSpec((blk, HID), lambda i: (i, 0)),
            pl.BlockSpec((1, HID), lambda i: (0, 0)),
            pl.BlockSpec((HID, 1), lambda i: (0, 0)),
            pl.BlockSpec((1, 1), lambda i: (0, 0)),
        ],
        out_specs=pl.BlockSpec((blk, 1), lambda i: (i, 0)),
        out_shape=jax.ShapeDtypeStruct((B, 1), jnp.float32),
    )(psum, b1.reshape(1, HID), W2, b2.reshape(1, 1))


def kernel(x, table, W1, b1, W2, b2):
    xr = x.astype(jnp.int32).reshape(B * 2, HALF_L)
    t1 = _fold_table(table, W1)
    psum = _sc_pool(xr, t1)
    return _tc_mlp(psum, b1, W2, b2)
